# Initial kernel scaffold; baseline (speedup 1.0000x reference)
import jax
import jax.numpy as jnp
from jax.experimental import pallas as pl


def kernel(feature, edge_index, W1, b1, W2, b2, Wl, bl):
    raise NotImplementedError("write your pallas kernel here")


# speedup vs baseline: 8.1142x; 8.1142x over previous
import functools
import jax
import jax.numpy as jnp
from jax import lax
from jax.experimental import pallas as pl
from jax.experimental.pallas import tpu as pltpu
from jax.experimental.pallas import tpu_sc as plsc

N = 10000
IN_DIM = 128
HID_DIM = 512
OUT_DIM = 768
E = 320000

NC = 2
NS = 16
NW = NC * NS
B = 128
NB = -(-E // (B * NW))
EP = NB * B * NW
NPAD = 10112
ROWS_PER_TILE = NPAD // NS

BN = 512
GRID = -(-NPAD // BN)

@functools.cache
def _mesh():
    return plsc.VectorSubcoreMesh(core_axis_name="c", subcore_axis_name="s",
                                  num_cores=NC, num_subcores=NS)


def _deg_body(zeros_hbm, ones_hbm, dst_hbm, out_hbm, didx, rows, acc, sem):
    c = lax.axis_index("c")
    s = lax.axis_index("s")
    wid = s * NC + c
    r0 = s * ROWS_PER_TILE
    pltpu.sync_copy(zeros_hbm.at[pl.ds(r0, ROWS_PER_TILE)],
                    acc.at[pl.ds(r0, ROWS_PER_TILE)])
    pltpu.sync_copy(ones_hbm, rows)
    plsc.subcore_barrier()
    base = wid * (NB * B)

    @pl.loop(0, NB)
    def _(j):
        off = base + j * B
        pltpu.sync_copy(dst_hbm.at[pl.ds(off, B)], didx)
        pltpu.sync_copy(rows, acc.at[didx], add=True)

    plsc.subcore_barrier()
    pltpu.sync_copy(acc.at[pl.ds(r0, ROWS_PER_TILE)],
                    out_hbm.at[c, pl.ds(r0, ROWS_PER_TILE)])


def _prop_body(zeros_hbm, src_hbm, dst_hbm, table_hbm, out_hbm,
               sidx, didx, rows, acc, sem):
    c = lax.axis_index("c")
    s = lax.axis_index("s")
    wid = s * NC + c
    r0 = s * ROWS_PER_TILE
    pltpu.sync_copy(zeros_hbm.at[pl.ds(r0, ROWS_PER_TILE)],
                    acc.at[pl.ds(r0, ROWS_PER_TILE)])
    plsc.subcore_barrier()
    base = wid * (NB * B)

    @pl.loop(0, NB)
    def _(j):
        off = base + j * B
        pltpu.sync_copy(src_hbm.at[pl.ds(off, B)], sidx)
        pltpu.sync_copy(dst_hbm.at[pl.ds(off, B)], didx)
        pltpu.async_copy(table_hbm.at[sidx], rows, sem).wait()
        pltpu.sync_copy(rows, acc.at[didx], add=True)

    plsc.subcore_barrier()
    pltpu.sync_copy(acc.at[pl.ds(r0, ROWS_PER_TILE)],
                    out_hbm.at[c, pl.ds(r0, ROWS_PER_TILE)])


@functools.cache
def _make_deg(interpret=False):
    return pl.kernel(
        _deg_body,
        out_type=jax.ShapeDtypeStruct((NC, NPAD, IN_DIM), jnp.float32),
        mesh=_mesh(),
        scratch_types=[
            pltpu.VMEM((B,), jnp.int32),
            pltpu.VMEM((B, IN_DIM), jnp.float32),
            pltpu.VMEM_SHARED((NPAD, IN_DIM), jnp.float32),
            pltpu.SemaphoreType.DMA,
        ],
        interpret=interpret,
    )


@functools.cache
def _make_prop(interpret=False):
    return pl.kernel(
        _prop_body,
        out_type=jax.ShapeDtypeStruct((NC, NPAD, IN_DIM), jnp.float32),
        mesh=_mesh(),
        scratch_types=[
            pltpu.VMEM((B,), jnp.int32),
            pltpu.VMEM((B,), jnp.int32),
            pltpu.VMEM((B, IN_DIM), jnp.float32),
            pltpu.VMEM_SHARED((NPAD, IN_DIM), jnp.float32),
            pltpu.SemaphoreType.DMA,
        ],
        interpret=interpret,
    )


def _scale_body(deg2_ref, x_ref, y1_ref, dinv_ref):
    deg = deg2_ref[0, :, 0:1] + deg2_ref[1, :, 0:1] + 1.0
    dinv = lax.rsqrt(deg)
    y1_ref[...] = x_ref[...] * dinv
    dinv_ref[...] = jnp.broadcast_to(dinv, (BN, IN_DIM))


def _make_scale(interpret=False):
    return pl.pallas_call(
        _scale_body,
        grid=(GRID,),
        in_specs=[
            pl.BlockSpec((NC, BN, IN_DIM), lambda i: (0, i, 0)),
            pl.BlockSpec((BN, IN_DIM), lambda i: (i, 0)),
        ],
        out_specs=[
            pl.BlockSpec((BN, IN_DIM), lambda i: (i, 0)),
            pl.BlockSpec((BN, IN_DIM), lambda i: (i, 0)),
        ],
        out_shape=[
            jax.ShapeDtypeStruct((NPAD, IN_DIM), jnp.float32),
            jax.ShapeDtypeStruct((NPAD, IN_DIM), jnp.float32),
        ],
        interpret=interpret,
    )


def _mm1_body(acc_ref, y1_ref, dinv_ref, w1_ref, b1_ref, y2_ref):
    dinv = dinv_ref[...]
    sx = (acc_ref[0] + acc_ref[1] + y1_ref[...]) * dinv
    h = jnp.dot(sx, w1_ref[...], preferred_element_type=jnp.float32,
                precision=lax.Precision.HIGHEST)
    h = jnp.maximum(h + b1_ref[...], 0.0)
    y2 = h * dinv[:, 0:1]
    for ck in range(4):
        y2_ref[ck] = y2[:, ck * IN_DIM:(ck + 1) * IN_DIM]


def _make_mm1(interpret=False):
    return pl.pallas_call(
        _mm1_body,
        grid=(GRID,),
        in_specs=[
            pl.BlockSpec((NC, BN, IN_DIM), lambda i: (0, i, 0)),
            pl.BlockSpec((BN, IN_DIM), lambda i: (i, 0)),
            pl.BlockSpec((BN, IN_DIM), lambda i: (i, 0)),
            pl.BlockSpec((IN_DIM, HID_DIM), lambda i: (0, 0)),
            pl.BlockSpec((1, HID_DIM), lambda i: (0, 0)),
        ],
        out_specs=pl.BlockSpec((4, BN, IN_DIM), lambda i: (0, i, 0)),
        out_shape=jax.ShapeDtypeStruct((4, NPAD, IN_DIM), jnp.float32),
        interpret=interpret,
    )


def _mm2_body(a0_ref, a1_ref, a2_ref, a3_ref, y2_ref, dinv_ref, x_ref,
              w2_ref, wl_ref, b2_ref, bl_ref, out_ref):
    dinv = dinv_ref[...]
    accs = (a0_ref, a1_ref, a2_ref, a3_ref)
    h2 = jnp.broadcast_to(b2_ref[...], (BN, OUT_DIM))
    for ck in range(4):
        sx = (accs[ck][0] + accs[ck][1] + y2_ref[ck]) * dinv
        h2 = h2 + jnp.dot(sx, w2_ref[ck * IN_DIM:(ck + 1) * IN_DIM, :],
                          preferred_element_type=jnp.float32,
                          precision=lax.Precision.HIGHEST)
    out = jnp.dot(h2, wl_ref[...], preferred_element_type=jnp.float32,
                  precision=lax.Precision.HIGHEST)
    out = jnp.maximum(out + bl_ref[...], 0.0)
    out_ref[:, 0:IN_DIM] = x_ref[...]
    out_ref[:, IN_DIM:] = out


def _make_mm2(interpret=False):
    return pl.pallas_call(
        _mm2_body,
        grid=(GRID,),
        in_specs=[
            pl.BlockSpec((NC, BN, IN_DIM), lambda i: (0, i, 0)),
            pl.BlockSpec((NC, BN, IN_DIM), lambda i: (0, i, 0)),
            pl.BlockSpec((NC, BN, IN_DIM), lambda i: (0, i, 0)),
            pl.BlockSpec((NC, BN, IN_DIM), lambda i: (0, i, 0)),
            pl.BlockSpec((4, BN, IN_DIM), lambda i: (0, i, 0)),
            pl.BlockSpec((BN, IN_DIM), lambda i: (i, 0)),
            pl.BlockSpec((BN, IN_DIM), lambda i: (i, 0)),
            pl.BlockSpec((HID_DIM, OUT_DIM), lambda i: (0, 0)),
            pl.BlockSpec((OUT_DIM, OUT_DIM), lambda i: (0, 0)),
            pl.BlockSpec((1, OUT_DIM), lambda i: (0, 0)),
            pl.BlockSpec((1, OUT_DIM), lambda i: (0, 0)),
        ],
        out_specs=pl.BlockSpec((BN, IN_DIM + OUT_DIM), lambda i: (i, 0)),
        out_shape=jax.ShapeDtypeStruct((NPAD, IN_DIM + OUT_DIM), jnp.float32),
        interpret=interpret,
    )


_scale_k = _make_scale()
_mm1_k = _make_mm1()
_mm2_k = _make_mm2()


@jax.jit
def kernel(feature, edge_index, W1, b1, W2, b2, Wl, bl):
    _deg_k = _make_deg()
    _prop_k = _make_prop()
    src = edge_index[0].astype(jnp.int32)
    dst = edge_index[1].astype(jnp.int32)
    pad = jnp.full((EP - E,), N, jnp.int32)
    srcp = jnp.concatenate([src, pad])
    dstp = jnp.concatenate([dst, pad])
    xpad = jnp.pad(feature, ((0, NPAD - N), (0, 0)))
    zeros128 = jnp.zeros((NPAD, IN_DIM), jnp.float32)
    ones128 = jnp.ones((B, IN_DIM), jnp.float32)

    deg2 = _deg_k(zeros128, ones128, dstp)
    y1, dinv128 = _scale_k(deg2, xpad)
    acc1 = _prop_k(zeros128, srcp, dstp, y1)
    y2_4 = _mm1_k(acc1, y1, dinv128, W1, b1.reshape(1, HID_DIM))
    accs = [_prop_k(zeros128, srcp, dstp, y2_4[ck]) for ck in range(4)]
    out = _mm2_k(accs[0], accs[1], accs[2], accs[3], y2_4, dinv128, xpad,
                 W2, Wl, b2.reshape(1, OUT_DIM), bl.reshape(1, OUT_DIM))
    return out[:N]
